# baseline (speedup 1.0000x reference)
import functools

import jax
import jax.numpy as jnp
from jax import lax
from jax.experimental import pallas as pl
from jax.experimental.pallas import tpu as pltpu
from jax.experimental.pallas import tpu_sc as plsc

VOCAB = 1000000
DIM = 32
B = 4096
L = 200

_R = 5
_W = 128
_TP = 131

_NBLK = VOCAB // _W
_TAIL = VOCAB - _NBLK * _W


def _make_relayout():
    info = plsc.get_sparse_core_info()
    nc = info.num_cores
    nw = nc * info.num_subcores
    sb = 4 * _W
    nsb = _NBLK // 4
    per_w = nsb // nw
    rest = nsb - per_w * nw + 1

    mesh = plsc.VectorSubcoreMesh(core_axis_name="c", subcore_axis_name="s")

    @functools.partial(
        pl.kernel,
        mesh=mesh,
        out_type=jax.ShapeDtypeStruct((VOCAB // 4, _W), jnp.float32),
        scratch_types=(
            [pltpu.VMEM((DIM, sb), jnp.float32) for _ in range(2)]
            + [pltpu.VMEM((sb // 4, _TP), jnp.float32) for _ in range(2)]
            + [pltpu.SemaphoreType.DMA] * 4
        ),
        compiler_params=pltpu.CompilerParams(use_tc_tiling_on_sc=True,
                                             needs_layout_passes=False),
    )
    def k(wt_hbm, tail_hbm, o_hbm, *refs):
        wvm = refs[0:2]
        obuf = refs[2:4]
        sem_l = refs[4:6]
        sem_s = refs[6:8]

        wid = lax.axis_index("s") * nc + lax.axis_index("c")
        iota = lax.iota(jnp.int32, 16)

        def blk(i):
            return wid + nw * i

        def src(i):
            return wt_hbm.at[:, pl.ds(blk(i) * sb, sb)]

        def dst(i):
            return o_hbm.at[pl.ds(blk(i) * (sb // 4), sb // 4), :]

        def l_start(i, s):
            pltpu.async_copy(src(i), wvm[s], sem_l[s])

        def l_wait(i, s):
            pltpu.make_async_copy(src(i), wvm[s], sem_l[s]).wait()

        def obuf_src(s):
            return obuf[s].at[:, pl.ds(0, _W)]

        def s_start(i, s):
            pltpu.async_copy(obuf_src(s), dst(i), sem_s[s])

        def s_wait(i, s):
            pltpu.make_async_copy(obuf_src(s), dst(i), sem_s[s]).wait()

        p_vec = [[(b * _W + g * 16 + iota) // 4 for g in range(_W // 16)]
                 for b in range(4)]
        q_vec = [((g * 16 + iota) % 4) * DIM for g in range(_W // 16)]

        def transpose(s):
            def per_d(d, carry):
                for b in range(4):
                    for g in range(_W // 16):
                        v = wvm[s][d, pl.ds(b * _W + g * 16, 16)]
                        plsc.store_scatter(obuf[s],
                                           [p_vec[b][g], q_vec[g] + d], v)
                return carry

            lax.fori_loop(0, DIM, per_d, 0, unroll=2)

        def step(i, s, wait_store, start_load):
            l_wait(i, s)
            if wait_store:
                s_wait(i - 2, s)
            transpose(s)
            if start_load:
                l_start(i + 2, s)
            s_start(i, s)

        l_start(0, 0)
        l_start(1, 1)
        step(0, 0, False, True)
        step(1, 1, False, True)

        def pair(gp, carry):
            step(gp * 2, 0, True, True)
            step(gp * 2 + 1, 1, True, True)
            return carry

        lax.fori_loop(1, (per_w - 3) // 2, pair, 0)

        step(per_w - 3, 0, True, True)
        step(per_w - 2, 1, True, False)
        step(per_w - 1, 0, True, False)
        s_wait(per_w - 2, 1)
        s_wait(per_w - 1, 0)

        lo = (nsb - 1) * sb

        @pl.when(wid == 0)
        def _():
            lsrc = wt_hbm.at[:, pl.ds(lo, sb)]
            pltpu.async_copy(lsrc, wvm[0], sem_l[0])
            pltpu.make_async_copy(lsrc, wvm[0], sem_l[0]).wait()
            transpose(0)
            ldst = o_hbm.at[pl.ds(lo // 4, sb // 4), :]
            pltpu.async_copy(obuf_src(0), ldst, sem_s[0])
            pltpu.make_async_copy(obuf_src(0), ldst, sem_s[0]).wait()

        @pl.when(wid == 1)
        def _():
            tvm = obuf[0].at[pl.ds(0, 16), pl.ds(0, _W)]
            pltpu.async_copy(tail_hbm, tvm, sem_l[0])
            pltpu.make_async_copy(tail_hbm, tvm, sem_l[0]).wait()
            pltpu.async_copy(tvm, o_hbm.at[pl.ds(_NBLK * DIM, 16), :],
                             sem_s[0])
            pltpu.make_async_copy(tvm,
                                  o_hbm.at[pl.ds(_NBLK * DIM, 16), :],
                                  sem_s[0]).wait()

    return k


def _make_gather():
    info = plsc.get_sparse_core_info()
    nc = info.num_cores
    nw = nc * info.num_subcores
    assert nw * _W == B and L % _R == 0

    mesh = plsc.VectorSubcoreMesh(core_axis_name="c", subcore_axis_name="s")

    @functools.partial(
        pl.kernel,
        mesh=mesh,
        out_type=jax.ShapeDtypeStruct((L, DIM // 8, B // _W, 8, _W),
                                      jnp.float32),
        scratch_types=(
            [pltpu.VMEM((L // 8, 8, _W), jnp.int32)]
            + [pltpu.VMEM((_W, DIM), jnp.float32) for _ in range(_R)]
            + [pltpu.VMEM((DIM // 8, 8, _TP), jnp.float32)
               for _ in range(_R)]
            + [pltpu.SemaphoreType.DMA] * (2 * _R + 1)
        ),
        compiler_params=pltpu.CompilerParams(use_tc_tiling_on_sc=False,
                                             needs_layout_passes=False),
    )
    def k(xq_hbm, tbl_hbm, out_hbm, *refs):
        idxs = refs[0]
        rows = refs[1:1 + _R]
        touts = refs[1 + _R:1 + 2 * _R]
        sem_g = refs[1 + 2 * _R:1 + 3 * _R]
        sem_s = refs[1 + 3 * _R:1 + 4 * _R]
        sem_i = refs[1 + 4 * _R]

        wid = lax.axis_index("s") * nc + lax.axis_index("c")

        pltpu.async_copy(xq_hbm.at[:, wid], idxs, sem_i).wait()

        iota = lax.iota(jnp.int32, 16)

        def idx_ref(l):
            return idxs.at[l // 8, l % 8]

        def g_start(l, s):
            pltpu.async_copy(tbl_hbm.at[idx_ref(l)], rows[s], sem_g[s])

        def g_wait(l, s):
            pltpu.make_async_copy(tbl_hbm.at[idx_ref(l)], rows[s],
                                  sem_g[s]).wait()

        def out_slice(l):
            return out_hbm.at[l, :, wid]

        def tout_src(s):
            return touts[s].at[:, :, pl.ds(0, _W)]

        def s_start(l, s):
            pltpu.async_copy(tout_src(s), out_slice(l), sem_s[s])

        def s_wait(l, s):
            pltpu.make_async_copy(tout_src(s), out_slice(l), sem_s[s]).wait()

        d_hi1, d_lo1 = iota // 8, iota % 8
        d_hi2, d_lo2 = (16 + iota) // 8, (16 + iota) % 8

        def transpose(s):
            def per_j(j, carry):
                jcol = jnp.full((16,), 0, jnp.int32) + j
                v1 = rows[s][j, pl.ds(0, 16)]
                v2 = rows[s][j, pl.ds(16, 16)]
                plsc.store_scatter(touts[s], [d_hi1, d_lo1, jcol], v1)
                plsc.store_scatter(touts[s], [d_hi2, d_lo2, jcol], v2)
                return carry

            lax.fori_loop(0, _W, per_j, 0, unroll=4)

        def step(l, s, wait_store, start_gather):
            g_wait(l, s)
            if start_gather:
                g_start(l + (_R - 1), (s + _R - 1) % _R)
            if wait_store:
                s_wait(l - _R, s)
            transpose(s)
            s_start(l, s)

        for s in range(_R - 1):
            g_start(s, s)
        for i in range(_R):
            step(i, i, False, True)

        def block(blk, carry):
            for i in range(_R):
                step(blk * _R + i, i, True, True)
            return carry

        lax.fori_loop(1, L // _R - 1, block, 0)

        for i in range(_R):
            l = L - _R + i
            step(l, i, True, i == 0)
        for i in range(_R):
            s_wait(L - _R + i, i)

    return k


_relayout = _make_relayout()
_gather = _make_gather()


@jax.jit
def kernel(x, weight):
    xq = (jnp.swapaxes(x.astype(jnp.int32), 0, 1)
          .reshape(L // 8, 8, B // _W, _W)
          .transpose(0, 2, 1, 3))
    tail = weight[_NBLK * _W:, :].reshape(16, _W)
    tbl = _relayout(jnp.swapaxes(weight, 0, 1), tail).reshape(VOCAB, DIM)
    o5 = _gather(xq, tbl)
    return (o5.transpose(0, 1, 3, 2, 4)
            .reshape(L, DIM, B)
            .transpose(2, 0, 1))


# speedup vs baseline: 1.1391x; 1.1391x over previous
import functools

import jax
import jax.numpy as jnp
from jax import lax
from jax.experimental import pallas as pl
from jax.experimental.pallas import tpu as pltpu
from jax.experimental.pallas import tpu_sc as plsc

VOCAB = 1000000
DIM = 32
B = 4096
L = 200

_R = 5
_W = 128
_TP = 131

_NBLK = VOCAB // _W
_TAIL = VOCAB - _NBLK * _W


def _make_relayout():
    info = plsc.get_sparse_core_info()
    nc = info.num_cores
    nw = nc * info.num_subcores
    ch = 256
    nch = VOCAB // ch
    per_w = nch // nw
    tail = VOCAB - nch * ch

    mesh = plsc.VectorSubcoreMesh(core_axis_name="c", subcore_axis_name="s")

    @functools.partial(
        pl.kernel,
        mesh=mesh,
        out_type=jax.ShapeDtypeStruct((VOCAB // 4, _W), jnp.float32),
        scratch_types=(
            [pltpu.VMEM((ch, DIM), jnp.float32) for _ in range(2)]
            + [pltpu.VMEM((ch // 4, _W), jnp.float32) for _ in range(2)]
            + [pltpu.SemaphoreType.DMA] * 4
        ),
        compiler_params=pltpu.CompilerParams(use_tc_tiling_on_sc=True,
                                             needs_layout_passes=False),
    )
    def k(tbl_hbm, o_hbm, *refs):
        vin = refs[0:2]
        vout = refs[2:4]
        sem_l = refs[4:6]
        sem_s = refs[6:8]

        wid = lax.axis_index("s") * nc + lax.axis_index("c")

        def blk(i):
            return wid + nw * i

        def src(i):
            return tbl_hbm.at[pl.ds(blk(i) * ch, ch), :]

        def dst(i):
            return o_hbm.at[pl.ds(blk(i) * (ch // 4), ch // 4), :]

        def l_start(i, s):
            pltpu.async_copy(src(i), vin[s], sem_l[s])

        def l_wait(i, s):
            pltpu.make_async_copy(src(i), vin[s], sem_l[s]).wait()

        def s_start(i, s):
            pltpu.async_copy(vout[s], dst(i), sem_s[s])

        def s_wait(i, s):
            pltpu.make_async_copy(vout[s], dst(i), sem_s[s]).wait()

        def repack(s, nlines):
            def per_line(j, carry):
                for q in range(4):
                    for h in range(2):
                        vout[s][j, pl.ds(q * DIM + h * 16, 16)] = (
                            vin[s][4 * j + q, pl.ds(h * 16, 16)])
                return carry

            lax.fori_loop(0, nlines, per_line, 0, unroll=2)

        def step(i, s, wait_store, start_load):
            l_wait(i, s)
            if wait_store:
                s_wait(i - 2, s)
            repack(s, ch // 4)
            if start_load:
                l_start(i + 2, s)
            s_start(i, s)

        l_start(0, 0)
        l_start(1, 1)
        step(0, 0, False, True)
        step(1, 1, False, True)

        def pair(gp, carry):
            step(gp * 2, 0, True, True)
            step(gp * 2 + 1, 1, True, True)
            return carry

        lax.fori_loop(1, (per_w - 2) // 2, pair, 0)

        step(per_w - 2, 0, True, False)
        step(per_w - 1, 1, True, False)
        s_wait(per_w - 2, 0)
        s_wait(per_w - 1, 1)

        for w in range(2):
            @pl.when(wid == w)
            def _(w=w):
                c = nw * per_w + w
                lsrc = tbl_hbm.at[pl.ds(c * ch, ch), :]
                pltpu.async_copy(lsrc, vin[0], sem_l[0])
                pltpu.make_async_copy(lsrc, vin[0], sem_l[0]).wait()
                repack(0, ch // 4)
                ldst = o_hbm.at[pl.ds(c * ch // 4, ch // 4), :]
                pltpu.async_copy(vout[0], ldst, sem_s[0])
                pltpu.make_async_copy(vout[0], ldst, sem_s[0]).wait()

        @pl.when(wid == 2)
        def _():
            tsrc = tbl_hbm.at[pl.ds(VOCAB - tail, tail), :]
            tin = vin[0].at[pl.ds(0, tail), :]
            pltpu.async_copy(tsrc, tin, sem_l[0])
            pltpu.make_async_copy(tsrc, tin, sem_l[0]).wait()
            repack(0, tail // 4)
            tdst = o_hbm.at[pl.ds((VOCAB - tail) // 4, tail // 4), :]
            tout = vout[0].at[pl.ds(0, tail // 4), :]
            pltpu.async_copy(tout, tdst, sem_s[0])
            pltpu.make_async_copy(tout, tdst, sem_s[0]).wait()

    return k


def _make_gather():
    info = plsc.get_sparse_core_info()
    nc = info.num_cores
    nw = nc * info.num_subcores
    assert nw * _W == B and L % _R == 0

    mesh = plsc.VectorSubcoreMesh(core_axis_name="c", subcore_axis_name="s")

    @functools.partial(
        pl.kernel,
        mesh=mesh,
        out_type=jax.ShapeDtypeStruct((L, DIM // 8, B // _W, 8, _W),
                                      jnp.float32),
        scratch_types=(
            [pltpu.VMEM((L // 8, 8, _W), jnp.int32)]
            + [pltpu.VMEM((_W, DIM), jnp.float32) for _ in range(_R)]
            + [pltpu.VMEM((DIM // 8, 8, _TP), jnp.float32)
               for _ in range(_R)]
            + [pltpu.SemaphoreType.DMA] * (2 * _R + 1)
        ),
        compiler_params=pltpu.CompilerParams(use_tc_tiling_on_sc=False,
                                             needs_layout_passes=False),
    )
    def k(xq_hbm, tbl_hbm, out_hbm, *refs):
        idxs = refs[0]
        rows = refs[1:1 + _R]
        touts = refs[1 + _R:1 + 2 * _R]
        sem_g = refs[1 + 2 * _R:1 + 3 * _R]
        sem_s = refs[1 + 3 * _R:1 + 4 * _R]
        sem_i = refs[1 + 4 * _R]

        wid = lax.axis_index("s") * nc + lax.axis_index("c")

        pltpu.async_copy(xq_hbm.at[:, wid], idxs, sem_i).wait()

        iota = lax.iota(jnp.int32, 16)

        def idx_ref(l):
            return idxs.at[l // 8, l % 8]

        def g_start(l, s):
            pltpu.async_copy(tbl_hbm.at[idx_ref(l)], rows[s], sem_g[s])

        def g_wait(l, s):
            pltpu.make_async_copy(tbl_hbm.at[idx_ref(l)], rows[s],
                                  sem_g[s]).wait()

        def out_slice(l):
            return out_hbm.at[l, :, wid]

        def tout_src(s):
            return touts[s].at[:, :, pl.ds(0, _W)]

        def s_start(l, s):
            pltpu.async_copy(tout_src(s), out_slice(l), sem_s[s])

        def s_wait(l, s):
            pltpu.make_async_copy(tout_src(s), out_slice(l), sem_s[s]).wait()

        d_hi1, d_lo1 = iota // 8, iota % 8
        d_hi2, d_lo2 = (16 + iota) // 8, (16 + iota) % 8

        def transpose(s):
            def per_j(j, carry):
                jcol = jnp.full((16,), 0, jnp.int32) + j
                v1 = rows[s][j, pl.ds(0, 16)]
                v2 = rows[s][j, pl.ds(16, 16)]
                plsc.store_scatter(touts[s], [d_hi1, d_lo1, jcol], v1)
                plsc.store_scatter(touts[s], [d_hi2, d_lo2, jcol], v2)
                return carry

            lax.fori_loop(0, _W, per_j, 0, unroll=4)

        def step(l, s, wait_store, start_gather):
            g_wait(l, s)
            if start_gather:
                g_start(l + (_R - 1), (s + _R - 1) % _R)
            if wait_store:
                s_wait(l - _R, s)
            transpose(s)
            s_start(l, s)

        for s in range(_R - 1):
            g_start(s, s)
        for i in range(_R):
            step(i, i, False, True)

        def block(blk, carry):
            for i in range(_R):
                step(blk * _R + i, i, True, True)
            return carry

        lax.fori_loop(1, L // _R - 1, block, 0)

        for i in range(_R):
            l = L - _R + i
            step(l, i, True, i == 0)
        for i in range(_R):
            s_wait(L - _R + i, i)

    return k


_relayout = _make_relayout()
_gather = _make_gather()


@jax.jit
def kernel(x, weight):
    xq = (jnp.swapaxes(x.astype(jnp.int32), 0, 1)
          .reshape(L // 8, 8, B // _W, _W)
          .transpose(0, 2, 1, 3))
    tbl = _relayout(weight).reshape(VOCAB, DIM)
    o5 = _gather(xq, tbl)
    return (o5.transpose(0, 1, 3, 2, 4)
            .reshape(L, DIM, B)
            .transpose(2, 0, 1))


# speedup vs baseline: 1.2348x; 1.0840x over previous
import functools

import jax
import jax.numpy as jnp
from jax import lax
from jax.experimental import pallas as pl
from jax.experimental.pallas import tpu as pltpu
from jax.experimental.pallas import tpu_sc as plsc

VOCAB = 1000000
DIM = 32
B = 4096
L = 200

_R = 5
_W = 128
_TP = 131


def _make_kernel():
    info = plsc.get_sparse_core_info()
    nc = info.num_cores
    nw = nc * info.num_subcores
    assert nw * _W == B and L % _R == 0

    mesh = plsc.VectorSubcoreMesh(core_axis_name="c", subcore_axis_name="s")

    @functools.partial(
        pl.kernel,
        mesh=mesh,
        out_type=jax.ShapeDtypeStruct((L, DIM // 8, B // _W, 8, _W),
                                      jnp.float32),
        scratch_types=(
            [pltpu.VMEM((L // 8, 8, _W), jnp.int32)]
            + [pltpu.VMEM((_W, DIM), jnp.float32) for _ in range(_R)]
            + [pltpu.VMEM((DIM // 8, 8, _TP), jnp.float32)
               for _ in range(_R)]
            + [pltpu.SemaphoreType.DMA] * (2 * _R + 1)
        ),
        compiler_params=pltpu.CompilerParams(use_tc_tiling_on_sc=False,
                                             needs_layout_passes=False),
    )
    def k(xq_hbm, tbl_hbm, out_hbm, *refs):
        idxs = refs[0]
        rows = refs[1:1 + _R]
        touts = refs[1 + _R:1 + 2 * _R]
        sem_g = refs[1 + 2 * _R:1 + 3 * _R]
        sem_s = refs[1 + 3 * _R:1 + 4 * _R]
        sem_i = refs[1 + 4 * _R]

        wid = lax.axis_index("s") * nc + lax.axis_index("c")

        pltpu.async_copy(xq_hbm.at[:, wid], idxs, sem_i).wait()

        iota = lax.iota(jnp.int32, 16)

        def idx_ref(l):
            return idxs.at[l // 8, l % 8]

        def g_start(l, s):
            pltpu.async_copy(tbl_hbm.at[idx_ref(l)], rows[s], sem_g[s])

        def g_wait(l, s):
            pltpu.make_async_copy(tbl_hbm.at[idx_ref(l)], rows[s],
                                  sem_g[s]).wait()

        def out_slice(l):
            return out_hbm.at[l, :, wid]

        def tout_src(s):
            return touts[s].at[:, :, pl.ds(0, _W)]

        def s_start(l, s):
            pltpu.async_copy(tout_src(s), out_slice(l), sem_s[s])

        def s_wait(l, s):
            pltpu.make_async_copy(tout_src(s), out_slice(l), sem_s[s]).wait()

        d_hi1, d_lo1 = iota // 8, iota % 8
        d_hi2, d_lo2 = (16 + iota) // 8, (16 + iota) % 8

        def transpose(s):
            def per_j(j, carry):
                jcol = jnp.full((16,), 0, jnp.int32) + j
                v1 = rows[s][j, pl.ds(0, 16)]
                v2 = rows[s][j, pl.ds(16, 16)]
                plsc.store_scatter(touts[s], [d_hi1, d_lo1, jcol], v1)
                plsc.store_scatter(touts[s], [d_hi2, d_lo2, jcol], v2)
                return carry

            lax.fori_loop(0, _W, per_j, 0, unroll=4)

        def step(l, s, wait_store, start_gather):
            g_wait(l, s)
            if start_gather:
                g_start(l + (_R - 1), (s + _R - 1) % _R)
            if wait_store:
                s_wait(l - _R, s)
            transpose(s)
            s_start(l, s)

        for s in range(_R - 1):
            g_start(s, s)
        for i in range(_R):
            step(i, i, False, True)

        def block(blk, carry):
            for i in range(_R):
                step(blk * _R + i, i, True, True)
            return carry

        lax.fori_loop(1, L // _R - 1, block, 0)

        for i in range(_R):
            l = L - _R + i
            step(l, i, True, i == 0)
        for i in range(_R):
            s_wait(L - _R + i, i)

    return k


_gather = _make_kernel()


@jax.jit
def kernel(x, weight):
    xq = (jnp.swapaxes(x.astype(jnp.int32), 0, 1)
          .reshape(L // 8, 8, B // _W, _W)
          .transpose(0, 2, 1, 3))
    o5 = _gather(xq, weight)
    return (o5.transpose(0, 1, 3, 2, 4)
            .reshape(L, DIM, B)
            .transpose(2, 0, 1))
